# baseline (speedup 1.0000x reference)
import functools

import jax
import jax.numpy as jnp
import numpy as np
from jax.experimental import pallas as pl
from jax.experimental.pallas import tpu as pltpu

_EPS = float(np.finfo(np.float32).tiny)
_K = 16
_TAU = 0.1
_REP = 2
_SHIFT = 4.0
_CLAMP = 85.0
_TOPC = 8
_BIGF = 3.0e38


def _body(s_ref, g_ref, o_ref, wz, kh, *, rep, nrow, n_iter, k):
    sub, lanes = s_ref.shape[1], s_ref.shape[2]
    nt = sub // 8
    reps = [(r, j) for r in range(rep) for j in range(nrow)]
    lane1 = jax.lax.broadcasted_iota(jnp.int32, (1, lanes), 1)
    srow8 = jax.lax.broadcasted_iota(jnp.int32, (8, lanes), 0)
    lane8 = jax.lax.broadcasted_iota(jnp.int32, (8, lanes), 1)

    nst = len(reps)
    cm8 = [jnp.full((8, lanes), -_BIGF, jnp.float32) for _ in reps]
    for v in range(nt):
        sv = [s_ref[j, pl.ds(v * 8, 8), :] for j in range(nrow)]
        for i, (r, j) in enumerate(reps):
            zv = sv[j] + g_ref[r, j, pl.ds(v * 8, 8), :]
            wz[i, pl.ds(v * 8, 8), :] = zv
            cm8[i] = jnp.maximum(cm8[i], zv)
    nv = [jnp.max(cm8[i]) - _SHIFT for i in range(nst)]
    s8 = [jnp.zeros((8, lanes), jnp.float32) for _ in reps]
    for v in range(nt):
        for i in range(nst):
            zv = wz[i, pl.ds(v * 8, 8), :]
            wv = jnp.exp(jnp.minimum((zv - nv[i]) * (1.0 / _TAU), _CLAMP))
            wz[i, pl.ds(v * 8, 8), :] = wv
            s8[i] = s8[i] + wv
    ssum = [jnp.sum(s8[i]) for i in range(nst)]
    for t in range(n_iter):
        rw = [1.0 / ssum[i] for i in range(nst)]
        s8 = [jnp.zeros((8, lanes), jnp.float32) for _ in reps]
        for v in range(nt):
            for i in range(nst):
                wv = wz[i, pl.ds(v * 8, 8), :]
                p = wv * rw[i]
                if t == 0:
                    kv = p
                else:
                    kv = kh[i, pl.ds(v * 8, 8), :] + p
                kh[i, pl.ds(v * 8, 8), :] = kv
                a = 1.0 - p
                a2 = a * a
                a4 = a2 * a2
                a8 = a4 * a4
                wn = wv * (a8 * a2)
                wz[i, pl.ds(v * 8, 8), :] = wn
                s8[i] = s8[i] + wn
        ssum = [jnp.sum(s8[i]) for i in range(nst)]
    thv = [jnp.full((1, lanes), _BIGF, jnp.float32) for _ in reps]
    ths = [jnp.full((1, lanes), -1, jnp.int32) for _ in reps]
    cand_v = [[] for _ in reps]
    cand_s = [[] for _ in reps]
    for j in range(_TOPC):
        best = [jnp.full((8, lanes), -1.0, jnp.float32) for _ in reps]
        bsr = [jnp.zeros((8, lanes), jnp.int32) for _ in reps]
        thv_b = [jnp.broadcast_to(thv[i], (8, lanes)) for i in range(nst)]
        ths_b = [jnp.broadcast_to(ths[i], (8, lanes)) for i in range(nst)]
        for v in range(nt):
            for i in range(nst):
                kv = kh[i, pl.ds(v * 8, 8), :]
                sr = srow8 + v * 8
                elig = jnp.logical_or(
                    kv < thv_b[i],
                    jnp.logical_and(kv == thv_b[i], sr > ths_b[i]))
                kk = jnp.where(elig, kv, -1.0)
                take = jnp.logical_or(
                    kk > best[i],
                    jnp.logical_and(kk == best[i], sr < bsr[i]))
                best[i] = jnp.where(take, kk, best[i])
                bsr[i] = jnp.where(take, sr, bsr[i])
        for i in range(nst):
            bv, bs = best[i], bsr[i]
            for h in (4, 2, 1):
                top_v, bot_v = bv[:h, :], bv[h:2 * h, :]
                top_s, bot_s = bs[:h, :], bs[h:2 * h, :]
                take = jnp.logical_or(
                    bot_v > top_v,
                    jnp.logical_and(bot_v == top_v, bot_s < top_s))
                bv = jnp.where(take, bot_v, top_v)
                bs = jnp.where(take, bot_s, top_s)
            thv[i], ths[i] = bv, bs
            cand_v[i].append(bv)
            cand_s[i].append(bs)
    cval = [jnp.concatenate(cand_v[i], axis=0) for i in range(nst)]
    csr = [jnp.concatenate(cand_s[i], axis=0) for i in range(nst)]
    cidx = [csr[i] * lanes + lane8[: _TOPC, :] for i in range(nst)]
    zero8 = jnp.zeros((8, lanes), jnp.float32)
    for v in range(nt):
        for (r, j) in reps:
            o_ref[r, j, pl.ds(v * 8, 8), :] = zero8
    work = list(cval)
    for _ in range(k):
        cm = [jnp.max(work[i]) for i in range(nst)]
        mi = []
        for i in range(nst):
            cnd = jnp.where(work[i] == cm[i], cidx[i], jnp.int32(sub * lanes))
            mi.append(jnp.min(cnd))
        for i, (r, j) in enumerate(reps):
            pick = jnp.logical_and(work[i] == cm[i], cidx[i] == mi[i])
            work[i] = jnp.where(pick, -1.0, work[i])
            val = (1.0 - cm[i]) + cm[i]
            row = mi[i] // lanes
            col = mi[i] - row * lanes
            cur = o_ref[r, j, pl.ds(row, 1), :]
            o_ref[r, j, pl.ds(row, 1), :] = cur + jnp.where(
                lane1 == col, val, 0.0)


def kernel(scores):
    bsz, nmax, ens = scores.shape
    rep = _REP
    k = min(_K, nmax)
    r1 = bsz * ens
    lanes = 128
    sub = nmax // lanes

    s2 = jnp.transpose(scores, (0, 2, 1)).reshape(r1, sub, lanes)
    gkey = jax.random.fold_in(jax.random.key(0), 1)
    g = jax.random.gumbel(gkey, (rep * r1, nmax), dtype=jnp.float32)
    g4 = g.reshape(rep, r1, sub, lanes)

    nrow = 16 if r1 % 16 == 0 else (2 if r1 % 2 == 0 else 1)
    res = pl.pallas_call(
        functools.partial(_body, rep=rep, nrow=nrow, n_iter=k, k=k),
        grid=(r1 // nrow,),
        in_specs=[
            pl.BlockSpec((nrow, sub, lanes), lambda i: (i, 0, 0)),
            pl.BlockSpec((rep, nrow, sub, lanes), lambda i: (0, i, 0, 0)),
        ],
        out_specs=pl.BlockSpec((rep, nrow, sub, lanes), lambda i: (0, i, 0, 0)),
        out_shape=jax.ShapeDtypeStruct((rep, r1, sub, lanes), jnp.float32),
        scratch_shapes=[
            pltpu.VMEM((rep * nrow, sub, lanes), jnp.float32),
            pltpu.VMEM((rep * nrow, sub, lanes), jnp.float32),
        ],
        compiler_params=pltpu.CompilerParams(
            dimension_semantics=("arbitrary",),
        ),
    )(s2, g4)

    return res.reshape(rep, bsz, ens, nmax).transpose(0, 1, 3, 2)


# speedup vs baseline: 1.1318x; 1.1318x over previous
import functools

import jax
import jax.numpy as jnp
import numpy as np
from jax.experimental import pallas as pl
from jax.experimental.pallas import tpu as pltpu

_EPS = float(np.finfo(np.float32).tiny)
_K = 16
_TAU = 0.1
_REP = 2
_SHIFT = 4.0
_CLAMP = 85.0
_TOPC = 6
_BIGF = 3.0e38


def _body(s_ref, g_ref, o_ref, wz, kh, *, rep, nrow, n_iter, k):
    sub, lanes = s_ref.shape[1], s_ref.shape[2]
    nt = sub // 8
    reps = [(r, j) for r in range(rep) for j in range(nrow)]
    lane1 = jax.lax.broadcasted_iota(jnp.int32, (1, lanes), 1)
    srow8 = jax.lax.broadcasted_iota(jnp.int32, (8, lanes), 0)
    lane8 = jax.lax.broadcasted_iota(jnp.int32, (8, lanes), 1)

    nst = len(reps)
    cm8 = [jnp.full((8, lanes), -_BIGF, jnp.float32) for _ in reps]
    for v in range(nt):
        sv = [s_ref[j, pl.ds(v * 8, 8), :] for j in range(nrow)]
        for i, (r, j) in enumerate(reps):
            zv = sv[j] + g_ref[r, j, pl.ds(v * 8, 8), :]
            wz[i, pl.ds(v * 8, 8), :] = zv
            cm8[i] = jnp.maximum(cm8[i], zv)
    nv = [jnp.max(cm8[i]) - _SHIFT for i in range(nst)]
    s8 = [jnp.zeros((8, lanes), jnp.float32) for _ in reps]
    for v in range(nt):
        for i in range(nst):
            zv = wz[i, pl.ds(v * 8, 8), :]
            wv = jnp.exp(jnp.minimum((zv - nv[i]) * (1.0 / _TAU), _CLAMP))
            wz[i, pl.ds(v * 8, 8), :] = wv
            s8[i] = s8[i] + wv
    ssum = [jnp.sum(s8[i]) for i in range(nst)]
    for t in range(n_iter):
        rw = [1.0 / ssum[i] for i in range(nst)]
        s8 = [jnp.zeros((8, lanes), jnp.float32) for _ in reps]
        for v in range(nt):
            for i in range(nst):
                wv = wz[i, pl.ds(v * 8, 8), :]
                p = wv * rw[i]
                if t == 0:
                    kv = p
                else:
                    kv = kh[i, pl.ds(v * 8, 8), :] + p
                kh[i, pl.ds(v * 8, 8), :] = kv
                a = 1.0 - p
                a2 = a * a
                a4 = a2 * a2
                a8 = a4 * a4
                wn = wv * (a8 * a2)
                wz[i, pl.ds(v * 8, 8), :] = wn
                s8[i] = s8[i] + wn
        ssum = [jnp.sum(s8[i]) for i in range(nst)]
    thv = [jnp.full((1, lanes), _BIGF, jnp.float32) for _ in reps]
    ths = [jnp.full((1, lanes), -1, jnp.int32) for _ in reps]
    cand_v = [[] for _ in reps]
    cand_s = [[] for _ in reps]
    for j in range(_TOPC):
        best = [jnp.full((8, lanes), -1.0, jnp.float32) for _ in reps]
        bsr = [jnp.zeros((8, lanes), jnp.int32) for _ in reps]
        thv_b = [jnp.broadcast_to(thv[i], (8, lanes)) for i in range(nst)]
        ths_b = [jnp.broadcast_to(ths[i], (8, lanes)) for i in range(nst)]
        for v in range(nt):
            for i in range(nst):
                kv = kh[i, pl.ds(v * 8, 8), :]
                sr = srow8 + v * 8
                elig = jnp.logical_or(
                    kv < thv_b[i],
                    jnp.logical_and(kv == thv_b[i], sr > ths_b[i]))
                kk = jnp.where(elig, kv, -1.0)
                take = jnp.logical_or(
                    kk > best[i],
                    jnp.logical_and(kk == best[i], sr < bsr[i]))
                best[i] = jnp.where(take, kk, best[i])
                bsr[i] = jnp.where(take, sr, bsr[i])
        for i in range(nst):
            bv, bs = best[i], bsr[i]
            for h in (4, 2, 1):
                top_v, bot_v = bv[:h, :], bv[h:2 * h, :]
                top_s, bot_s = bs[:h, :], bs[h:2 * h, :]
                take = jnp.logical_or(
                    bot_v > top_v,
                    jnp.logical_and(bot_v == top_v, bot_s < top_s))
                bv = jnp.where(take, bot_v, top_v)
                bs = jnp.where(take, bot_s, top_s)
            thv[i], ths[i] = bv, bs
            cand_v[i].append(bv)
            cand_s[i].append(bs)
    cval = [jnp.concatenate(cand_v[i], axis=0) for i in range(nst)]
    csr = [jnp.concatenate(cand_s[i], axis=0) for i in range(nst)]
    cidx = [csr[i] * lanes + lane8[: _TOPC, :] for i in range(nst)]
    zero8 = jnp.zeros((8, lanes), jnp.float32)
    for v in range(nt):
        for (r, j) in reps:
            o_ref[r, j, pl.ds(v * 8, 8), :] = zero8
    work = list(cval)
    for _ in range(k):
        cm = [jnp.max(work[i]) for i in range(nst)]
        mi = []
        for i in range(nst):
            cnd = jnp.where(work[i] == cm[i], cidx[i], jnp.int32(sub * lanes))
            mi.append(jnp.min(cnd))
        for i, (r, j) in enumerate(reps):
            pick = jnp.logical_and(work[i] == cm[i], cidx[i] == mi[i])
            work[i] = jnp.where(pick, -1.0, work[i])
            val = (1.0 - cm[i]) + cm[i]
            row = mi[i] // lanes
            col = mi[i] - row * lanes
            cur = o_ref[r, j, pl.ds(row, 1), :]
            o_ref[r, j, pl.ds(row, 1), :] = cur + jnp.where(
                lane1 == col, val, 0.0)


def kernel(scores):
    bsz, nmax, ens = scores.shape
    rep = _REP
    k = min(_K, nmax)
    r1 = bsz * ens
    lanes = 128
    sub = nmax // lanes

    s2 = jnp.transpose(scores, (0, 2, 1)).reshape(r1, sub, lanes)
    gkey = jax.random.fold_in(jax.random.key(0), 1)
    g = jax.random.gumbel(gkey, (rep * r1, nmax), dtype=jnp.float32)
    g4 = g.reshape(rep, r1, sub, lanes)

    nrow = 8 if r1 % 8 == 0 else (2 if r1 % 2 == 0 else 1)
    res = pl.pallas_call(
        functools.partial(_body, rep=rep, nrow=nrow, n_iter=k, k=k),
        grid=(r1 // nrow,),
        in_specs=[
            pl.BlockSpec((nrow, sub, lanes), lambda i: (i, 0, 0)),
            pl.BlockSpec((rep, nrow, sub, lanes), lambda i: (0, i, 0, 0)),
        ],
        out_specs=pl.BlockSpec((rep, nrow, sub, lanes), lambda i: (0, i, 0, 0)),
        out_shape=jax.ShapeDtypeStruct((rep, r1, sub, lanes), jnp.float32),
        scratch_shapes=[
            pltpu.VMEM((rep * nrow, sub, lanes), jnp.float32),
            pltpu.VMEM((rep * nrow, sub, lanes), jnp.float32),
        ],
        compiler_params=pltpu.CompilerParams(
            dimension_semantics=("arbitrary",),
        ),
    )(s2, g4)

    return res.reshape(rep, bsz, ens, nmax).transpose(0, 1, 3, 2)
